# baseline (speedup 1.0000x reference)
import functools

import jax
import jax.numpy as jnp
from jax import lax
from jax.experimental import pallas as pl
from jax.experimental.pallas import tpu as pltpu
from jax.experimental.pallas import tpu_sc as plsc

_D = 128
_E = 16
_LANES = 16
_CHUNKS = 2
_TILE = 128
_NW = 32
_SC_ROWS = 16384
_TC_BLK = 16384


def _tc_router_body(x_ref, w_ref, o_ref):
    logits = jnp.dot(x_ref[...], w_ref[...], preferred_element_type=jnp.float32)
    m = jnp.max(logits, axis=-1, keepdims=True)
    e = jnp.exp(logits - m)
    o_ref[...] = e / jnp.sum(e, axis=-1, keepdims=True)


def _make_sc_body(base_row):
    def _sc_router_body(x_hbm, ws_hbm, o_hbm, ws_v, xb_v, ob_v):
        strip = _SC_ROWS // _NW
        wid = lax.axis_index("s") * 2 + lax.axis_index("c")
        base = base_row + wid * strip
        obase_w = wid * strip
        pltpu.sync_copy(ws_hbm, ws_v)
        lanes = jnp.arange(_LANES, dtype=jnp.int32)

        def tile_body(t, carry):
            r0 = base + t * _TILE
            pltpu.sync_copy(x_hbm.at[pl.ds(r0, _TILE)], xb_v)
            for blk in range(_TILE // (_LANES * _CHUNKS)):
                rowsets = [
                    lanes + (blk * _CHUNKS + c) * _LANES
                    for c in range(_CHUNKS)
                ]

                def dbody(d, accs):
                    col = jnp.full((_LANES,), d, dtype=jnp.int32)
                    xT = [plsc.load_gather(xb_v, [rs, col]) for rs in rowsets]
                    return tuple(
                        accs[c * _E + e] + xT[c] * ws_v[d, e]
                        for c in range(_CHUNKS)
                        for e in range(_E)
                    )

                accs = lax.fori_loop(
                    0, _D, dbody,
                    tuple(jnp.zeros((_LANES,), jnp.float32)
                          for _ in range(_CHUNKS * _E)),
                )
                for c in range(_CHUNKS):
                    ac = accs[c * _E:(c + 1) * _E]
                    m = ac[0]
                    for e in range(1, _E):
                        m = jnp.maximum(m, ac[e])
                    es = [jnp.exp(a - m) for a in ac]
                    s = es[0]
                    for e in range(1, _E):
                        s = s + es[e]
                    r = 1.0 / s
                    ob = lanes + (blk * _CHUNKS + c) * _LANES
                    for e in range(_E):
                        plsc.store_scatter(
                            ob_v,
                            [ob, jnp.full((_LANES,), e, dtype=jnp.int32)],
                            es[e] * r)
            pltpu.sync_copy(
                ob_v, o_hbm.at[pl.ds(obase_w + t * _TILE, _TILE)])
            return carry

        lax.fori_loop(0, strip // _TILE, tile_body, 0)

    return _sc_router_body


def _sc_router(x2flat, wsplat, base_row):
    mesh = plsc.VectorSubcoreMesh(core_axis_name="c", subcore_axis_name="s")
    f = pl.kernel(
        _make_sc_body(base_row),
        mesh=mesh,
        out_type=jax.ShapeDtypeStruct((_SC_ROWS, _E), jnp.float32),
        compiler_params=pltpu.CompilerParams(
            needs_layout_passes=False, use_tc_tiling_on_sc=False),
        scratch_types=[
            pltpu.VMEM((_D, _E, _LANES), jnp.float32),
            pltpu.VMEM((_TILE, _D), jnp.float32),
            pltpu.VMEM((_TILE, _E), jnp.float32),
        ],
    )
    return f(x2flat, wsplat)


def kernel(x, expert_embeddings):
    B, S, H, D = x.shape
    E = expert_embeddings.shape[1]
    R = B * S * H
    r_tc = R - _SC_ROWS

    wsplat = jnp.broadcast_to(
        expert_embeddings.reshape(D, E, 1), (D, E, _LANES)
    )
    x2 = x.reshape(R, D)
    out_sc = _sc_router(x2, wsplat, r_tc)
    out_tc = pl.pallas_call(
        _tc_router_body,
        grid=(r_tc // _TC_BLK,),
        in_specs=[
            pl.BlockSpec((_TC_BLK, D), lambda i: (i, 0)),
            pl.BlockSpec((D, E), lambda i: (0, 0)),
        ],
        out_specs=pl.BlockSpec((_TC_BLK, E), lambda i: (i, 0)),
        out_shape=jax.ShapeDtypeStruct((R, E), jnp.float32),
    )(x2, expert_embeddings)

    out = lax.dynamic_update_slice(out_tc, out_sc, (r_tc, 0))
    return out.reshape(B, S, H, E)


# speedup vs baseline: 1.7383x; 1.7383x over previous
import jax
import jax.numpy as jnp
from jax.experimental import pallas as pl

_SBLK = 1024


def _tc_router_body(x_ref, w_ref, o_ref):
    sb, h, d = x_ref.shape[1], x_ref.shape[2], x_ref.shape[3]
    e = w_ref.shape[1]
    x2 = x_ref[...].reshape(sb * h, d)
    logits = jnp.dot(x2, w_ref[...], preferred_element_type=jnp.float32)
    m = jnp.max(logits, axis=-1, keepdims=True)
    ex = jnp.exp(logits - m)
    res = ex / jnp.sum(ex, axis=-1, keepdims=True)
    o_ref[...] = res.reshape(1, sb, h, e)


def kernel(x, expert_embeddings):
    B, S, H, D = x.shape
    E = expert_embeddings.shape[1]
    return pl.pallas_call(
        _tc_router_body,
        grid=(B, S // _SBLK),
        in_specs=[
            pl.BlockSpec((1, _SBLK, H, D), lambda b, s: (b, s, 0, 0)),
            pl.BlockSpec((D, E), lambda b, s: (0, 0)),
        ],
        out_specs=pl.BlockSpec((1, _SBLK, H, E), lambda b, s: (b, s, 0, 0)),
        out_shape=jax.ShapeDtypeStruct((B, S, H, E), jnp.float32),
    )(x, expert_embeddings)
